# baseline (speedup 1.0000x reference)
import functools

import jax
import jax.numpy as jnp
from jax import lax
from jax.experimental import pallas as pl
from jax.experimental.pallas import tpu as pltpu
from jax.experimental.pallas import tpu_sc as plsc

B = 16384
D = 64
NC = 2
NS = 16
L = 16
NW = NC * NS
BPW = B // NW
C = 128
NCHUNKS = BPW // C
GROUPS = C // L


def _body(head_hbm, tail_hbm, rel_hbm, ere_hbm, eim_hbm, rre_hbm, rim_hbm,
          out_hbm,
          hidx, tidx, ridx, hre, him, tre, tim, rre, rim, outv, sem):
    wid = lax.axis_index("s") * NC + lax.axis_index("c")
    base = wid * BPW

    pltpu.sync_copy(head_hbm.at[pl.ds(base, BPW)], hidx)
    pltpu.sync_copy(tail_hbm.at[pl.ds(base, BPW)], tidx)
    pltpu.sync_copy(rel_hbm.at[pl.ds(base, BPW)], ridx)

    def chunk_body(ci, carry):
        off = ci * C
        cps = [
            pltpu.async_copy(ere_hbm.at[hidx.at[pl.ds(off, C)]], hre, sem),
            pltpu.async_copy(eim_hbm.at[hidx.at[pl.ds(off, C)]], him, sem),
            pltpu.async_copy(ere_hbm.at[tidx.at[pl.ds(off, C)]], tre, sem),
            pltpu.async_copy(eim_hbm.at[tidx.at[pl.ds(off, C)]], tim, sem),
            pltpu.async_copy(rre_hbm.at[ridx.at[pl.ds(off, C)]], rre, sem),
            pltpu.async_copy(rim_hbm.at[ridx.at[pl.ds(off, C)]], rim, sem),
        ]
        for cp in cps:
            cp.wait()

        def grp_body(g, carry2):
            rows = g * L + lax.iota(jnp.int32, L)

            def d_body(d, acc):
                col = jnp.full((L,), d, dtype=jnp.int32)
                a_hre = plsc.load_gather(hre, [rows, col])
                a_him = plsc.load_gather(him, [rows, col])
                a_tre = plsc.load_gather(tre, [rows, col])
                a_tim = plsc.load_gather(tim, [rows, col])
                a_rre = plsc.load_gather(rre, [rows, col])
                a_rim = plsc.load_gather(rim, [rows, col])
                sym = a_hre * a_tre + a_him * a_tim
                asym = a_hre * a_tim - a_him * a_tre
                return acc + (a_rre * sym + a_rim * asym)

            acc = lax.fori_loop(0, D, d_body, jnp.zeros((L,), jnp.float32))
            outv[pl.ds(g * L, L)] = 1.0 / (1.0 + jnp.exp(-acc))
            return carry2

        lax.fori_loop(0, GROUPS, grp_body, 0)
        pltpu.sync_copy(outv, out_hbm.at[pl.ds(base + off, C)])
        return carry

    lax.fori_loop(0, NCHUNKS, chunk_body, 0)


@jax.jit
def _run(head, tail, relation, entity_re, entity_im, relation_re, relation_im):
    f = pl.kernel(
        _body,
        out_type=jax.ShapeDtypeStruct((B,), jnp.float32),
        mesh=plsc.VectorSubcoreMesh(core_axis_name="c", subcore_axis_name="s"),
        compiler_params=pltpu.CompilerParams(
            needs_layout_passes=False, use_tc_tiling_on_sc=False),
        scratch_types=[
            pltpu.VMEM((BPW,), jnp.int32),
            pltpu.VMEM((BPW,), jnp.int32),
            pltpu.VMEM((BPW,), jnp.int32),
            pltpu.VMEM((C, D), jnp.float32),
            pltpu.VMEM((C, D), jnp.float32),
            pltpu.VMEM((C, D), jnp.float32),
            pltpu.VMEM((C, D), jnp.float32),
            pltpu.VMEM((C, D), jnp.float32),
            pltpu.VMEM((C, D), jnp.float32),
            pltpu.VMEM((C,), jnp.float32),
            pltpu.SemaphoreType.DMA,
        ],
    )
    return f(head, tail, relation, entity_re, entity_im, relation_re,
             relation_im)


def kernel(head, tail, relation, entity_re, entity_im, relation_re,
           relation_im):
    return _run(head, tail, relation, entity_re, entity_im, relation_re,
                relation_im)
